# baseline (speedup 1.0000x reference)
import functools

import jax
import jax.numpy as jnp
import numpy as _np
from jax import lax
from jax.experimental import pallas as pl
from jax.experimental.pallas import tpu as pltpu
from jax.experimental.pallas import tpu_sc as plsc

_IN = 50000
_OUT = 32
_F = 16
_NW = 32
_CH = 64
_NB = 3
_NH = 2
_PROBE_DMA_ONLY = False
_PROBE_NO_DMA = True


def _make_pass():
    mesh = plsc.VectorSubcoreMesh(core_axis_name="c", subcore_axis_name="s")

    @functools.partial(
        pl.kernel,
        mesh=mesh,
        compiler_params=pltpu.CompilerParams(
            needs_layout_passes=False, use_tc_tiling_on_sc=False),
        out_type=jax.ShapeDtypeStruct((_NW, _OUT, _F), jnp.float32),
        scratch_types=[
            pltpu.VMEM((_NB * _CH * _OUT, _F), jnp.float32),
            pltpu.VMEM((_OUT, _F), jnp.float32),
            pltpu.VMEM((_OUT, _F), jnp.float32),
            pltpu.SemaphoreType.DMA,
        ],
    )
    def sc_pass(u_hbm, vdiag_hbm, out_hbm, ubuf, vdiagv, sdiag, sem):
        cid = lax.axis_index("c")
        sid = lax.axis_index("s")
        w = sid * 2 + cid
        start = (w * _IN) // _NW
        end = ((w + 1) * _IN) // _NW
        count = end - start
        nchunks = (count + _CH - 1) // _CH

        iota = lax.iota(jnp.int32, _F)
        cols = [lax.rem(iota + c, _F) for c in range(_F)]
        zeros16 = jnp.zeros((_F,), jnp.float32)

        pltpu.sync_copy(vdiag_hbm, vdiagv)
        vd = [vdiagv[r, :] for r in range(_OUT)]
        for r in range(_OUT):
            sdiag[r, :] = zeros16

        def chunk_start(k):
            g = start + k * _CH
            d = jnp.minimum(g, end - _CH)
            slot = lax.rem(k, _NB)
            pltpu.make_async_copy(
                u_hbm.at[pl.ds(d * _OUT, _CH * _OUT)],
                ubuf.at[pl.ds(slot * _CH * _OUT, _CH * _OUT)],
                sem,
            ).start()

        if not _PROBE_NO_DMA:
            for k in range(_NB - 1):
                chunk_start(jnp.int32(k))

        def chunk_body(k, carry):
            if not _PROBE_NO_DMA:
                @pl.when(k + (_NB - 1) < nchunks)
                def _():
                    chunk_start(k + (_NB - 1))
                pltpu.make_async_copy(
                    u_hbm.at[pl.ds(0, _CH * _OUT)],
                    ubuf.at[pl.ds(0, _CH * _OUT)],
                    sem,
                ).wait()
            g = start + k * _CH
            d = jnp.minimum(g, end - _CH)
            lo = g - d
            srow = lax.rem(k, _NB) * (_CH * _OUT)

            def one_node(n):
                nrow = srow + n * _OUT
                rows = [jnp.full((_F,), nrow + h * _F, jnp.int32) + iota
                        for h in range(_NH)]
                cs = []
                for h in range(_NH):
                    accs = [None] * 4
                    for c in range(_F):
                        gv = plsc.load_gather(ubuf, [rows[h], cols[c]])
                        t = gv * vd[h * _F + c]
                        a = accs[c % 4]
                        accs[c % 4] = t if a is None else a + t
                    b = (accs[0] + accs[1]) + (accs[2] + accs[3])
                    cs.append(jnp.exp(b))
                ssum = jnp.sum(cs[0] + cs[1])
                rv = 1.0 / jnp.full((_F,), ssum, jnp.float32)
                return rows, [cs[0] * rv, cs[1] * rv]

            def accum_node(rows, cvecs):
                for h in range(_NH):
                    for c in range(_F):
                        gv = plsc.load_gather(ubuf, [rows[h], cols[c]])
                        plsc.addupdate(sdiag.at[h * _F + c], gv * cvecs[h])

            def node_body(n, c2):
                rows, cvecs = one_node(n)
                accum_node(rows, cvecs)
                return c2

            def pair_body(i, c2):
                n = lo2 + i * 2
                ra, ca = one_node(n)
                rb, cb = one_node(n + 1)
                accum_node(ra, ca)
                accum_node(rb, cb)
                return c2

            rem2 = lax.rem(_CH - lo, 2)
            lo2 = lo + rem2

            if _PROBE_DMA_ONLY:
                plsc.addupdate(sdiag.at[0], ubuf[srow, :])
                return carry

            @pl.when(rem2 == 1)
            def _():
                node_body(lo, 0)

            lax.fori_loop(0, (_CH - lo2) // 2, pair_body, 0)
            return carry

        lax.fori_loop(0, nchunks, chunk_body, 0)
        pltpu.sync_copy(sdiag, out_hbm.at[w])

    return sc_pass


_sc_pass = _make_pass()


_J = _np.arange(_OUT)[:, None]
_FG = _np.arange(_F)[None, :]


def _diag_pack(vacc):
    h = _J // _F
    c = _J % _F
    j = _FG
    return vacc[h * _F + j, (j + c) % _F]


def _diag_unpack(sd):
    jmod = _J % _F
    return sd[(_J // _F) * _F + (_FG - jmod) % _F, jmod]


def _squash_v(s):
    sq = jnp.sum(s ** 2, axis=1, keepdims=True)
    return sq / (1.0 + sq) * (s / jnp.sqrt(sq))


def kernel(u_hat, routing_num):
    def body(_, carry):
        vacc, _v = carry
        parts = _sc_pass(u_hat, _diag_pack(vacc))
        s = _diag_unpack(jnp.sum(parts, axis=0))
        v = _squash_v(s)
        return (vacc + v, v)

    init = (jnp.zeros((_OUT, _F), jnp.float32),
            jnp.zeros((_OUT, _F), jnp.float32))
    _, v = lax.fori_loop(0, routing_num, body, init)
    return v


# speedup vs baseline: 1.1709x; 1.1709x over previous
import functools

import jax
import jax.numpy as jnp
import numpy as _np
from jax import lax
from jax.experimental import pallas as pl
from jax.experimental.pallas import tpu as pltpu
from jax.experimental.pallas import tpu_sc as plsc

_IN = 50000
_OUT = 32
_F = 16
_NW = 32
_CH = 64
_NB = 3
_NH = 2


def _make_pass():
    mesh = plsc.VectorSubcoreMesh(core_axis_name="c", subcore_axis_name="s")

    @functools.partial(
        pl.kernel,
        mesh=mesh,
        compiler_params=pltpu.CompilerParams(
            needs_layout_passes=False, use_tc_tiling_on_sc=False),
        out_type=jax.ShapeDtypeStruct((_NW, _OUT, _F), jnp.float32),
        scratch_types=[
            pltpu.VMEM((_NB * _CH * _OUT, _F), jnp.float32),
            pltpu.VMEM((_OUT, _F), jnp.float32),
            pltpu.VMEM((_OUT, _F), jnp.float32),
            pltpu.SemaphoreType.DMA,
        ],
    )
    def sc_pass(u_hbm, vdiag_hbm, out_hbm, ubuf, vdiagv, sdiag, sem):
        cid = lax.axis_index("c")
        sid = lax.axis_index("s")
        w = sid * 2 + cid
        start = (w * _IN) // _NW
        end = ((w + 1) * _IN) // _NW
        count = end - start
        nchunks = (count + _CH - 1) // _CH

        iota = lax.iota(jnp.int32, _F)
        cols = [lax.rem(iota + c, _F) for c in range(_F)]
        zeros16 = jnp.zeros((_F,), jnp.float32)

        pltpu.sync_copy(vdiag_hbm, vdiagv)
        vd = [vdiagv[r, :] for r in range(_OUT)]
        for r in range(_OUT):
            sdiag[r, :] = zeros16

        def chunk_start(k):
            g = start + k * _CH
            d = jnp.minimum(g, end - _CH)
            slot = lax.rem(k, _NB)
            pltpu.make_async_copy(
                u_hbm.at[pl.ds(d * _OUT, _CH * _OUT)],
                ubuf.at[pl.ds(slot * _CH * _OUT, _CH * _OUT)],
                sem,
            ).start()

        for k in range(_NB - 1):
            chunk_start(jnp.int32(k))

        def chunk_body(k, carry):
            @pl.when(k + (_NB - 1) < nchunks)
            def _():
                chunk_start(k + (_NB - 1))
            pltpu.make_async_copy(
                u_hbm.at[pl.ds(0, _CH * _OUT)],
                ubuf.at[pl.ds(0, _CH * _OUT)],
                sem,
            ).wait()
            g = start + k * _CH
            d = jnp.minimum(g, end - _CH)
            lo = g - d
            srow = lax.rem(k, _NB) * (_CH * _OUT)

            def one_node(n):
                nrow = srow + n * _OUT
                rows = [jnp.full((_F,), nrow + h * _F, jnp.int32) + iota
                        for h in range(_NH)]
                cs = []
                for h in range(_NH):
                    accs = [None] * 4
                    for c in range(_F):
                        gv = plsc.load_gather(ubuf, [rows[h], cols[c]])
                        t = gv * vd[h * _F + c]
                        a = accs[c % 4]
                        accs[c % 4] = t if a is None else a + t
                    b = (accs[0] + accs[1]) + (accs[2] + accs[3])
                    cs.append(jnp.exp(b))
                ssum = jnp.sum(cs[0] + cs[1])
                rv = 1.0 / jnp.full((_F,), ssum, jnp.float32)
                return rows, [cs[0] * rv, cs[1] * rv]

            def accum_node(rows, cvecs):
                for h in range(_NH):
                    for c in range(_F):
                        gv = plsc.load_gather(ubuf, [rows[h], cols[c]])
                        plsc.addupdate(sdiag.at[h * _F + c], gv * cvecs[h])

            @plsc.parallel_loop(lo, _CH, unroll=4)
            def _node_loop(n):
                rows, cvecs = one_node(n)
                accum_node(rows, cvecs)

            return carry

        lax.fori_loop(0, nchunks, chunk_body, 0)
        pltpu.sync_copy(sdiag, out_hbm.at[w])

    return sc_pass


_sc_pass = _make_pass()


_J = _np.arange(_OUT)[:, None]
_FG = _np.arange(_F)[None, :]


def _diag_pack(vacc):
    h = _J // _F
    c = _J % _F
    j = _FG
    return vacc[h * _F + j, (j + c) % _F]


def _diag_unpack(sd):
    jmod = _J % _F
    return sd[(_J // _F) * _F + (_FG - jmod) % _F, jmod]


def _squash_v(s):
    sq = jnp.sum(s ** 2, axis=1, keepdims=True)
    return sq / (1.0 + sq) * (s / jnp.sqrt(sq))


def kernel(u_hat, routing_num):
    def body(_, carry):
        vacc, _v = carry
        parts = _sc_pass(u_hat, _diag_pack(vacc))
        s = _diag_unpack(jnp.sum(parts, axis=0))
        v = _squash_v(s)
        return (vacc + v, v)

    init = (jnp.zeros((_OUT, _F), jnp.float32),
            jnp.zeros((_OUT, _F), jnp.float32))
    _, v = lax.fori_loop(0, routing_num, body, init)
    return v
